# baseline (speedup 1.0000x reference)
import functools

import jax
import jax.numpy as jnp
from jax import lax
from jax.experimental import pallas as pl
from jax.experimental.pallas import tpu as pltpu, tpu_sc as plsc

B = 256
N_IN = 65536
N_OUT = 65536

NC = 2
NS = 16
NW = NC * NS
L = 16

ROWS_PER_W = B // NW
ICH = 8192
N_ICH = N_OUT // ICH
RES_ICH = 4
STR_ICH = N_ICH - RES_ICH
OCH = 4096
N_OCH = N_OUT // OCH
ODEPTH = 3


def _sampler_body(act_hbm, pidx_hbm, out_hbm, row_buf, idx_shared,
                  idx_res, idx_bufs, out_bufs, act_sems, idx_sems, out_sems):
    sid = lax.axis_index("s")
    wid = sid * NC + lax.axis_index("c")
    row0 = wid * ROWS_PER_W

    @pl.when(sid == 0)
    def _():
        pltpu.sync_copy(pidx_hbm, idx_shared)
    plsc.subcore_barrier()
    pltpu.sync_copy(idx_shared.at[pl.ds(0, RES_ICH * ICH)], idx_res)

    def compute_chunk(idx_buf, idx_off, out_buf):
        @plsc.parallel_loop(0, OCH // L, unroll=8)
        def grp(j):
            off = pl.multiple_of(j * L, L)
            packed = idx_buf[pl.ds(idx_off + off, L)]
            i0 = lax.bitwise_and(packed, jnp.int32(0xFFFF))
            i1 = lax.shift_right_logical(packed, jnp.int32(16))
            s0 = plsc.load_gather(row_buf, [i0])
            s1 = plsc.load_gather(row_buf, [i1])
            out_buf[pl.ds(off, L)] = 1.0 - s0 * s1

    oc_per_ich = ICH // OCH

    ACT_SPLIT = 4
    ACT_PIECE = N_IN // ACT_SPLIT

    def row_body(r, carry):
        row = row0 + r
        act_hs = [
            pltpu.async_copy(
                act_hbm.at[row, pl.ds(a * ACT_PIECE, ACT_PIECE)],
                row_buf.at[pl.ds(a * ACT_PIECE, ACT_PIECE)],
                act_sems[a])
            for a in range(ACT_SPLIT)
        ]
        idx_h = [None] * N_ICH
        out_h = [None] * N_OCH
        for k in range(min(2, STR_ICH)):
            idx_h[RES_ICH + k] = pltpu.async_copy(
                idx_shared.at[pl.ds((RES_ICH + k) * ICH, ICH)],
                idx_bufs[k % 2], idx_sems[k % 2])
        for h in act_hs:
            h.wait()
        for c in range(N_OCH):
            p = c % ODEPTH
            ci = c // oc_per_ich
            sub = (c % oc_per_ich) * OCH
            if c >= ODEPTH:
                out_h[c - ODEPTH].wait()
            if ci < RES_ICH:
                compute_chunk(idx_res, ci * ICH + sub, out_bufs[p])
            else:
                k = ci - RES_ICH
                if sub == 0:
                    idx_h[ci].wait()
                compute_chunk(idx_bufs[k % 2], sub, out_bufs[p])
                if sub != 0 and k + 2 < STR_ICH:
                    idx_h[ci + 2] = pltpu.async_copy(
                        idx_shared.at[pl.ds((ci + 2) * ICH, ICH)],
                        idx_bufs[k % 2], idx_sems[k % 2])
            out_h[c] = pltpu.async_copy(
                out_bufs[p], out_hbm.at[row, pl.ds(c * OCH, OCH)],
                out_sems[p])
        for c in range(N_OCH - ODEPTH, N_OCH):
            out_h[c].wait()
        return carry

    lax.fori_loop(0, ROWS_PER_W, row_body, 0)


@functools.partial(
    pl.kernel,
    out_type=jax.ShapeDtypeStruct((B, N_OUT), jnp.float32),
    mesh=plsc.VectorSubcoreMesh(core_axis_name="c", subcore_axis_name="s"),
    scratch_types=[
        pltpu.VMEM((N_IN,), jnp.float32),
        pltpu.VMEM_SHARED((N_OUT,), jnp.int32),
        pltpu.VMEM((RES_ICH * ICH,), jnp.int32),
        pltpu.VMEM((ICH,), jnp.int32),
        pltpu.VMEM((ICH,), jnp.int32),
        pltpu.VMEM((OCH,), jnp.float32),
        pltpu.VMEM((OCH,), jnp.float32),
        pltpu.VMEM((OCH,), jnp.float32),
        pltpu.SemaphoreType.DMA,
        pltpu.SemaphoreType.DMA,
        pltpu.SemaphoreType.DMA,
        pltpu.SemaphoreType.DMA,
        pltpu.SemaphoreType.DMA,
        pltpu.SemaphoreType.DMA,
        pltpu.SemaphoreType.DMA,
        pltpu.SemaphoreType.DMA,
        pltpu.SemaphoreType.DMA,
    ],
    compiler_params=pltpu.CompilerParams(needs_layout_passes=False),
)
def _sampler_kernel(act_hbm, pidx_hbm, out_hbm, row_buf, idx_shared,
                    idx_res, idx_a, idx_b, out_a, out_b, out_c,
                    sem_a0, sem_a1, sem_a2, sem_a3, sem_i0, sem_i1,
                    sem_o0, sem_o1, sem_o2):
    _sampler_body(act_hbm, pidx_hbm, out_hbm, row_buf, idx_shared,
                  idx_res, (idx_a, idx_b), (out_a, out_b, out_c),
                  (sem_a0, sem_a1, sem_a2, sem_a3), (sem_i0, sem_i1),
                  (sem_o0, sem_o1, sem_o2))


def kernel(activations, sample_indices):
    idx = sample_indices.astype(jnp.int32)
    packed = jnp.bitwise_or(idx[:, 0], jnp.left_shift(idx[:, 1], 16))
    return _sampler_kernel(activations, packed)


# speedup vs baseline: 1.0257x; 1.0257x over previous
import functools

import jax
import jax.numpy as jnp
from jax import lax
from jax.experimental import pallas as pl
from jax.experimental.pallas import tpu as pltpu, tpu_sc as plsc

B = 256
N_IN = 65536
N_OUT = 65536

NC = 2
NS = 16
NW = NC * NS
L = 16

ROWS_PER_W = B // NW
CHUNK = 8192
N_CHUNKS = N_OUT // CHUNK
DEPTH = 3


def _sampler_body(act_hbm, pidx_hbm, out_hbm, row_buf, idx_shared,
                  idx_bufs, out_bufs, sem_act, idx_sems, out_sems):
    sid = lax.axis_index("s")
    wid = sid * NC + lax.axis_index("c")
    row0 = wid * ROWS_PER_W

    @pl.when(sid == 0)
    def _():
        pltpu.sync_copy(pidx_hbm, idx_shared)
    plsc.subcore_barrier()

    def compute_chunk(idx_buf, out_buf):
        @plsc.parallel_loop(0, CHUNK // L, unroll=8)
        def grp(j):
            off = pl.multiple_of(j * L, L)
            packed = idx_buf[pl.ds(off, L)]
            i0 = lax.bitwise_and(packed, jnp.int32(0xFFFF))
            i1 = lax.shift_right_logical(packed, jnp.int32(16))
            s0 = plsc.load_gather(row_buf, [i0])
            s1 = plsc.load_gather(row_buf, [i1])
            out_buf[pl.ds(off, L)] = 1.0 - s0 * s1

    def row_body(r, carry):
        row = row0 + r
        act_h = pltpu.async_copy(act_hbm.at[row], row_buf, sem_act)
        idx_h = [None] * N_CHUNKS
        out_h = [None] * N_CHUNKS
        for k in range(DEPTH):
            idx_h[k] = pltpu.async_copy(
                idx_shared.at[pl.ds(k * CHUNK, CHUNK)],
                idx_bufs[k % DEPTH], idx_sems[k % DEPTH])
        act_h.wait()
        for c in range(N_CHUNKS):
            p = c % DEPTH
            idx_h[c].wait()
            if c >= DEPTH:
                out_h[c - DEPTH].wait()
            compute_chunk(idx_bufs[p], out_bufs[p])
            if c + DEPTH < N_CHUNKS:
                idx_h[c + DEPTH] = pltpu.async_copy(
                    idx_shared.at[pl.ds((c + DEPTH) * CHUNK, CHUNK)],
                    idx_bufs[p], idx_sems[p])
            out_h[c] = pltpu.async_copy(
                out_bufs[p], out_hbm.at[row, pl.ds(c * CHUNK, CHUNK)],
                out_sems[p])
        for c in range(N_CHUNKS - DEPTH, N_CHUNKS):
            out_h[c].wait()
        return carry

    lax.fori_loop(0, ROWS_PER_W, row_body, 0)


@functools.partial(
    pl.kernel,
    out_type=jax.ShapeDtypeStruct((B, N_OUT), jnp.float32),
    mesh=plsc.VectorSubcoreMesh(core_axis_name="c", subcore_axis_name="s"),
    scratch_types=[
        pltpu.VMEM((N_IN,), jnp.float32),
        pltpu.VMEM_SHARED((N_OUT,), jnp.int32),
        pltpu.VMEM((CHUNK,), jnp.int32),
        pltpu.VMEM((CHUNK,), jnp.int32),
        pltpu.VMEM((CHUNK,), jnp.int32),
        pltpu.VMEM((CHUNK,), jnp.float32),
        pltpu.VMEM((CHUNK,), jnp.float32),
        pltpu.VMEM((CHUNK,), jnp.float32),
        pltpu.SemaphoreType.DMA,
        pltpu.SemaphoreType.DMA,
        pltpu.SemaphoreType.DMA,
        pltpu.SemaphoreType.DMA,
        pltpu.SemaphoreType.DMA,
        pltpu.SemaphoreType.DMA,
        pltpu.SemaphoreType.DMA,
    ],
    compiler_params=pltpu.CompilerParams(needs_layout_passes=False),
)
def _sampler_kernel(act_hbm, pidx_hbm, out_hbm, row_buf, idx_shared,
                    idx_a, idx_b, idx_c, out_a, out_b, out_c,
                    sem_act, sem_i0, sem_i1, sem_i2,
                    sem_o0, sem_o1, sem_o2):
    _sampler_body(act_hbm, pidx_hbm, out_hbm, row_buf, idx_shared,
                  (idx_a, idx_b, idx_c), (out_a, out_b, out_c),
                  sem_act, (sem_i0, sem_i1, sem_i2),
                  (sem_o0, sem_o1, sem_o2))


def kernel(activations, sample_indices):
    idx = sample_indices.astype(jnp.int32)
    packed = jnp.bitwise_or(idx[:, 0], jnp.left_shift(idx[:, 1], 16))
    return _sampler_kernel(activations, packed)
